# baseline (speedup 1.0000x reference)
import functools

import jax
import jax.numpy as jnp
from jax import lax
from jax.experimental import pallas as pl
from jax.experimental.pallas import tpu as pltpu
from jax.experimental.pallas import tpu_sc as plsc

NC = 2
NS = 16
NW = NC * NS
NPOS = 4096
T = 200
D = 64
LB = 128
NT = NPOS // LB
NTASK = T * NT
TPW = NTASK // NW
NBUF = 2

_mesh = plsc.VectorSubcoreMesh(
    core_axis_name="c", subcore_axis_name="s", num_cores=NC, num_subcores=NS
)


@functools.partial(
    pl.kernel,
    out_type=jax.ShapeDtypeStruct((T, D // 8, NT, 8, LB), jnp.float32),
    mesh=_mesh,
    scratch_types=[
        pltpu.VMEM((TPW, LB), jnp.int32),
        [pltpu.VMEM((LB, D), jnp.float32) for _ in range(NBUF)],
        [pltpu.VMEM((D, LB), jnp.float32) for _ in range(NBUF)],
        [pltpu.SemaphoreType.DMA for _ in range(NBUF)],
        [pltpu.SemaphoreType.DMA for _ in range(NBUF)],
    ],
    compiler_params=pltpu.CompilerParams(
        use_tc_tiling_on_sc=False, needs_layout_passes=False
    ),
)
def _gather_t(table_hbm, idx_hbm, out_hbm, idx_v, bufa, bufb, gsem, wsem):
    wid = lax.axis_index("s") * NC + lax.axis_index("c")
    k0 = wid * TPW
    pltpu.sync_copy(idx_hbm.at[pl.ds(k0, TPW)], idx_v)

    def g_start(j, b):
        pltpu.async_copy(table_hbm.at[idx_v.at[j]], bufa[b], gsem[b])

    def g_wait(b):
        pltpu.make_async_copy(
            table_hbm.at[idx_v.at[0]], bufa[b], gsem[b]
        ).wait()

    def w_start(k, b):
        t = k // NT
        ih = k % NT
        for ch in range(D // 8):
            pltpu.async_copy(
                bufb[b].at[pl.ds(ch * 8, 8)], out_hbm.at[t, ch, ih], wsem[b]
            )

    def w_wait(b):
        for ch in range(D // 8):
            pltpu.make_async_copy(
                bufb[b].at[pl.ds(0, 8)], out_hbm.at[0, 0, 0], wsem[b]
            ).wait()

    row_ids = [
        lax.iota(jnp.int32, 16) + (blk * 16) for blk in range(LB // 16)
    ]

    def transpose(b):
        a, bb = bufa[b], bufb[b]

        @plsc.parallel_loop(0, D, 1, unroll=8)
        def _(d):
            dv = jnp.full((16,), d, jnp.int32)
            for blk in range(LB // 16):
                il = row_ids[blk]
                cv = (dv + il) & (D - 1)
                v = plsc.load_gather(a, [il, cv])
                plsc.store_scatter(bb, [cv, il], v)

    g_start(0, 0)
    g_start(1, 1)

    def body(i, carry):
        for b in range(NBUF):
            j = i * NBUF + b
            g_wait(b)

            @pl.when(j >= NBUF)
            def _():
                w_wait(b)

            transpose(b)
            w_start(k0 + j, b)

            @pl.when(j < TPW - NBUF)
            def _():
                g_start(j + NBUF, b)

        return carry

    lax.fori_loop(0, TPW // NBUF, body, 0)
    for b in range(NBUF):
        w_wait(b)


def kernel(embeddings, pos_ids):
    idx_t = pos_ids.T.reshape(NTASK, LB)
    out5 = _gather_t(embeddings, idx_t)
    return jnp.transpose(out5, (2, 4, 0, 1, 3)).reshape(NPOS, T, D)


# speedup vs baseline: 1.2204x; 1.2204x over previous
import functools

import jax
import jax.numpy as jnp
from jax import lax
from jax.experimental import pallas as pl
from jax.experimental.pallas import tpu as pltpu
from jax.experimental.pallas import tpu_sc as plsc

NC = 2
NS = 16
NW = NC * NS
NPOS = 4096
T = 200
D = 64
LB = 128
NT = NPOS // LB
NTASK = T * NT
TPW = NTASK // NW
NBUF = 4

_mesh = plsc.VectorSubcoreMesh(
    core_axis_name="c", subcore_axis_name="s", num_cores=NC, num_subcores=NS
)


@functools.partial(
    pl.kernel,
    out_type=jax.ShapeDtypeStruct((T, D // 8, NT, 8, LB), jnp.float32),
    mesh=_mesh,
    scratch_types=[
        pltpu.VMEM((TPW, LB), jnp.int32),
        [pltpu.VMEM((LB, D), jnp.float32) for _ in range(NBUF)],
        [pltpu.VMEM((D, LB), jnp.float32) for _ in range(NBUF)],
        [pltpu.SemaphoreType.DMA for _ in range(NBUF)],
        [pltpu.SemaphoreType.DMA for _ in range(NBUF)],
    ],
    compiler_params=pltpu.CompilerParams(
        use_tc_tiling_on_sc=False, needs_layout_passes=False
    ),
)
def _gather_t(table_hbm, idx_hbm, out_hbm, idx_v, bufa, bufb, gsem, wsem):
    wid = lax.axis_index("s") * NC + lax.axis_index("c")
    k0 = wid * TPW
    pltpu.sync_copy(idx_hbm.at[pl.ds(k0, TPW)], idx_v)

    def g_start(j, b):
        pltpu.async_copy(table_hbm.at[idx_v.at[j]], bufa[b], gsem[b])

    def g_wait(b):
        pltpu.make_async_copy(
            table_hbm.at[idx_v.at[0]], bufa[b], gsem[b]
        ).wait()

    def w_start(k, b):
        t = k // NT
        ih = k % NT
        for ch in range(D // 8):
            pltpu.async_copy(
                bufb[b].at[pl.ds(ch * 8, 8)], out_hbm.at[t, ch, ih], wsem[b]
            )

    def w_wait(b):
        for ch in range(D // 8):
            pltpu.make_async_copy(
                bufb[b].at[pl.ds(0, 8)], out_hbm.at[0, 0, 0], wsem[b]
            ).wait()

    row_ids = [
        lax.iota(jnp.int32, 16) + (blk * 16) for blk in range(LB // 16)
    ]

    def transpose(b):
        a, bb = bufa[b], bufb[b]

        @plsc.parallel_loop(0, D, 1, unroll=8)
        def _(d):
            dv = jnp.full((16,), d, jnp.int32)
            for blk in range(LB // 16):
                il = row_ids[blk]
                cv = (dv + il) & (D - 1)
                v = plsc.load_gather(a, [il, cv])
                plsc.store_scatter(bb, [cv, il], v)

    for b in range(NBUF):
        g_start(b, b)

    def body(i, carry):
        for b in range(NBUF):
            j = i * NBUF + b
            g_wait(b)

            @pl.when(j >= NBUF)
            def _():
                w_wait(b)

            transpose(b)
            w_start(k0 + j, b)

            @pl.when(j < TPW - NBUF)
            def _():
                g_start(j + NBUF, b)

        return carry

    lax.fori_loop(0, TPW // NBUF, body, 0)
    for b in range(NBUF):
        w_wait(b)


def kernel(embeddings, pos_ids):
    idx_t = pos_ids.T.reshape(NTASK, LB)
    out5 = _gather_t(embeddings, idx_t)
    return jnp.transpose(out5, (2, 4, 0, 1, 3)).reshape(NPOS, T, D)


# speedup vs baseline: 1.2212x; 1.0007x over previous
import functools

import jax
import jax.numpy as jnp
from jax import lax
from jax.experimental import pallas as pl
from jax.experimental.pallas import tpu as pltpu
from jax.experimental.pallas import tpu_sc as plsc

NC = 2
NS = 16
NW = NC * NS
NPOS = 4096
T = 200
D = 64
LB = 128
NT = NPOS // LB
NTASK = T * NT
TPW = NTASK // NW
NBUF = 4

_mesh = plsc.VectorSubcoreMesh(
    core_axis_name="c", subcore_axis_name="s", num_cores=NC, num_subcores=NS
)


@functools.partial(
    pl.kernel,
    out_type=jax.ShapeDtypeStruct((T, D // 8, NT, 8, LB), jnp.float32),
    mesh=_mesh,
    scratch_types=[
        pltpu.VMEM((TPW, LB), jnp.int32),
        [pltpu.VMEM((LB, D), jnp.float32) for _ in range(NBUF)],
        [pltpu.VMEM((D, LB), jnp.float32) for _ in range(NBUF)],
        [pltpu.SemaphoreType.DMA for _ in range(NBUF)],
        [pltpu.SemaphoreType.DMA for _ in range(NBUF)],
    ],
    compiler_params=pltpu.CompilerParams(
        use_tc_tiling_on_sc=False, needs_layout_passes=False
    ),
)
def _gather_t(table_hbm, idx_hbm, out_hbm, idx_v, bufa, bufb, gsem, wsem):
    wid = lax.axis_index("s") * NC + lax.axis_index("c")
    k0 = wid * TPW
    pltpu.sync_copy(idx_hbm.at[pl.ds(k0, TPW)], idx_v)

    def g_start(j, b):
        pltpu.async_copy(table_hbm.at[idx_v.at[j]], bufa[b], gsem[b])

    def g_wait(b):
        pltpu.make_async_copy(
            table_hbm.at[idx_v.at[0]], bufa[b], gsem[b]
        ).wait()

    def w_start(k, b):
        tb = k // (NT * 8)
        r = k % (NT * 8)
        ih = r // 8
        t = tb * 8 + r % 8
        for ch in range(D // 8):
            pltpu.async_copy(
                bufb[b].at[pl.ds(ch * 8, 8)], out_hbm.at[t, ch, ih], wsem[b]
            )

    def w_wait(b):
        for ch in range(D // 8):
            pltpu.make_async_copy(
                bufb[b].at[pl.ds(0, 8)], out_hbm.at[0, 0, 0], wsem[b]
            ).wait()

    row_ids = [
        lax.iota(jnp.int32, 16) + (blk * 16) for blk in range(LB // 16)
    ]

    def transpose(b):
        a, bb = bufa[b], bufb[b]

        @plsc.parallel_loop(0, D, 1, unroll=8)
        def _(d):
            dv = jnp.full((16,), d, jnp.int32)
            for blk in range(LB // 16):
                il = row_ids[blk]
                cv = (dv + il) & (D - 1)
                v = plsc.load_gather(a, [il, cv])
                plsc.store_scatter(bb, [cv, il], v)

    for b in range(NBUF):
        g_start(b, b)

    def body(i, carry):
        for b in range(NBUF):
            j = i * NBUF + b
            g_wait(b)

            @pl.when(j >= NBUF)
            def _():
                w_wait(b)

            transpose(b)
            w_start(k0 + j, b)

            @pl.when(j < TPW - NBUF)
            def _():
                g_start(j + NBUF, b)

        return carry

    lax.fori_loop(0, TPW // NBUF, body, 0)
    for b in range(NBUF):
        w_wait(b)


def kernel(embeddings, pos_ids):
    idx_t = (
        pos_ids.T.reshape(T // 8, 8, NT, LB)
        .transpose(0, 2, 1, 3)
        .reshape(NTASK, LB)
    )
    out5 = _gather_t(embeddings, idx_t)
    return jnp.transpose(out5, (2, 4, 0, 1, 3)).reshape(NPOS, T, D)
